# baseline (speedup 1.0000x reference)
import jax
import jax.numpy as jnp
from jax.experimental import pallas as pl
from jax.experimental.pallas import tpu as pltpu

N, NFEAT, NHID, NCLASS = 8192, 256, 64, 16

BLK = 256
NBLK = N // BLK
RESB = 3
NSLOT = 3


def _nt(lhs, rhs):
    return jax.lax.dot_general(lhs, rhs, (((1,), (1,)), ((), ())),
                               preferred_element_type=jnp.float32)


def _tn(lhs, rhs):
    return jax.lax.dot_general(lhs, rhs, (((0,), (0,)), ((), ())),
                               preferred_element_type=jnp.float32)


def _h1_kernel(x_ref, W1_ref, h1t_ref):
    h1t_ref[...] = jax.lax.dot_general(
        W1_ref[...], x_ref[...], (((0,), (1,)), ((), ())),
        preferred_element_type=jnp.float32).astype(jnp.bfloat16)


def _gcn_kernel(adj_ref, h1t_ref, W2_ref, W3_ref, b1_ref, b2_ref, b3_ref,
                linW_ref, linb_ref,
                adjb_hbm, out_ref,
                res_ref, rbuf_ref, x1t_ref, hcurt_ref, h3t_ref,
                acc1_ref, acc2_ref, acc3_ref, sems):
    i = pl.program_id(0)
    p = jax.lax.div(i, NBLK)
    j = jax.lax.rem(i, NBLK)

    def write_copy(b, slot):
        return pltpu.make_async_copy(
            rbuf_ref.at[slot],
            adjb_hbm.at[pl.ds(b * BLK, BLK), :],
            sems.at[slot])

    def read_copy(b, slot):
        return pltpu.make_async_copy(
            adjb_hbm.at[pl.ds(b * BLK, BLK), :],
            rbuf_ref.at[slot],
            sems.at[slot])

    def blockmax(acc_ref, v):
        @pl.when(j == 0)
        def _():
            acc_ref[...] = v

        @pl.when(j != 0)
        def _():
            acc_ref[...] = jnp.maximum(acc_ref[...], v)

    @pl.when(p == 0)
    def _():
        ab = adj_ref[...].astype(jnp.bfloat16)

        @pl.when(j < RESB)
        def _():
            res_ref[pl.ds(j * BLK, BLK), :] = ab

        @pl.when(j >= RESB)
        def _():
            slot = jax.lax.rem(j, NSLOT)

            @pl.when(j - NSLOT >= RESB)
            def _():
                write_copy(j - NSLOT, slot).wait()

            rbuf_ref[pl.ds(slot, 1)] = ab[None]
            write_copy(j, slot).start()

        yt = _nt(h1t_ref[...], ab)
        yrt = jnp.maximum(yt + b1_ref[...], 0.0)
        x1t_ref[:, pl.ds(j * BLK, BLK)] = yrt.astype(jnp.bfloat16)
        blockmax(acc1_ref, yrt)

    @pl.when(p > 0)
    def _():
        @pl.when(i == NBLK)
        def _():
            hcurt_ref[...] = _tn(W2_ref[...].astype(jnp.bfloat16),
                                 x1t_ref[...]).astype(jnp.bfloat16)

        @pl.when(i == 2 * NBLK)
        def _():
            hcurt_ref[...] = h3t_ref[...]

        b = j + 2
        @pl.when((b >= RESB) & (b < NBLK))
        def _():
            slot_b = jax.lax.rem(b, NSLOT)

            @pl.when((p == 1) & (b < RESB + NSLOT))
            def _():
                last_w = NBLK - 1 - jax.lax.rem(NBLK - 1 - slot_b, NSLOT)
                write_copy(last_w, slot_b).wait()

            read_copy(b, slot_b).start()

        def compute(abj):
            yt = _nt(hcurt_ref[...], abj)
            yb2 = jnp.maximum(yt + b2_ref[...], 0.0)

            @pl.when(p == 1)
            def _():
                h3t_ref[:, pl.ds(j * BLK, BLK)] = _tn(
                    W3_ref[...], yb2).astype(jnp.bfloat16)
                blockmax(acc2_ref, yb2)

            @pl.when(p == 2)
            def _():
                blockmax(acc3_ref, yt + b3_ref[...])

        @pl.when(j < RESB)
        def _():
            compute(res_ref[pl.ds(j * BLK, BLK), :])

        @pl.when(j >= RESB)
        def _():
            slot = jax.lax.rem(j, NSLOT)
            read_copy(j, slot).wait()
            compute(rbuf_ref[pl.ds(slot, 1)][0])

    @pl.when(i == 3 * NBLK - 1)
    def _():
        o1 = jnp.max(acc1_ref[...], axis=1, keepdims=True)
        o2 = jnp.max(acc2_ref[...], axis=1, keepdims=True)
        o3 = jnp.max(acc3_ref[...], axis=1, keepdims=True)
        lg = (jnp.dot(linW_ref[:, 0:NHID], o1,
                      preferred_element_type=jnp.float32)
              + jnp.dot(linW_ref[:, NHID:2 * NHID], o2,
                        preferred_element_type=jnp.float32)
              + jnp.dot(linW_ref[:, 2 * NHID:], o3,
                        preferred_element_type=jnp.float32)
              + linb_ref[...])
        z = lg - jnp.max(lg)
        out_ref[...] = z - jnp.log(jnp.sum(jnp.exp(z)))


def kernel(x, adj, W1, b1, W2, b2, W3, b3, linW, linb):
    full = lambda shape: pl.BlockSpec(shape, lambda i: (0, 0))

    h1t = pl.pallas_call(
        _h1_kernel,
        out_shape=jax.ShapeDtypeStruct((NHID, N), jnp.bfloat16),
    )(x, W1)

    _, out = pl.pallas_call(
        _gcn_kernel,
        grid=(3 * NBLK,),
        in_specs=[
            pl.BlockSpec((BLK, N), lambda i: (jnp.minimum(i, NBLK - 1), 0)),
            full((NHID, N)),
            full((NHID, NHID)),
            full((NHID, NHID)),
            full((NHID, 1)),
            full((NHID, 1)),
            full((NHID, 1)),
            full((NCLASS, 3 * NHID)),
            full((NCLASS, 1)),
        ],
        out_specs=[
            pl.BlockSpec(memory_space=pltpu.MemorySpace.HBM),
            pl.BlockSpec((NCLASS, 1), lambda i: (0, 0)),
        ],
        out_shape=[
            jax.ShapeDtypeStruct((N, N), jnp.bfloat16),
            jax.ShapeDtypeStruct((NCLASS, 1), jnp.float32),
        ],
        scratch_shapes=[
            pltpu.VMEM((RESB * BLK, N), jnp.bfloat16),
            pltpu.VMEM((NSLOT, BLK, N), jnp.bfloat16),
            pltpu.VMEM((NHID, N), jnp.bfloat16),
            pltpu.VMEM((NHID, N), jnp.bfloat16),
            pltpu.VMEM((NHID, N), jnp.bfloat16),
            pltpu.VMEM((NHID, BLK), jnp.float32),
            pltpu.VMEM((NHID, BLK), jnp.float32),
            pltpu.VMEM((NHID, BLK), jnp.float32),
            pltpu.SemaphoreType.DMA((NSLOT,)),
        ],
        compiler_params=pltpu.CompilerParams(
            dimension_semantics=("arbitrary",)),
    )(adj, h1t, W2, W3, b1.reshape(-1, 1), b2.reshape(-1, 1),
      b3.reshape(-1, 1), linW, linb.reshape(-1, 1))
    return out.reshape(NCLASS)


# speedup vs baseline: 1.0785x; 1.0785x over previous
import jax
import jax.numpy as jnp
from jax.experimental import pallas as pl
from jax.experimental.pallas import tpu as pltpu

N, NFEAT, NHID, NCLASS = 8192, 256, 64, 16

BLKA = 256
NBLKA = N // BLKA
BLKB = 1024
NBLKB = N // BLKB


def _layer1_kernel(adj_ref, x_ref, W1_ref, b1_ref,
                   adjb_ref, x1_ref, o1_ref, h1_ref, acc_ref):
    j = pl.program_id(0)

    @pl.when(j == 0)
    def _():
        h1_ref[...] = jnp.dot(x_ref[...], W1_ref[...],
                              preferred_element_type=jnp.float32
                              ).astype(jnp.bfloat16)

    ab = adj_ref[...].astype(jnp.bfloat16)
    adjb_ref[...] = ab
    y = jnp.dot(ab, h1_ref[...], preferred_element_type=jnp.float32)
    yr = jnp.maximum(y + b1_ref[...], 0.0)
    x1_ref[...] = yr.astype(jnp.bfloat16)

    @pl.when(j == 0)
    def _():
        acc_ref[...] = yr

    @pl.when(j != 0)
    def _():
        acc_ref[...] = jnp.maximum(acc_ref[...], yr)

    @pl.when(j == NBLKA - 1)
    def _():
        o1_ref[...] = jnp.max(acc_ref[...], axis=0, keepdims=True)


def _layer23_kernel(adjb_ref, x1_ref, W2_ref, W3_ref, b2_ref, b3_ref,
                    linW_ref, linb_ref, o1_ref, out_ref,
                    hcur_ref, h3_ref, acc2_ref, acc3_ref):
    i = pl.program_id(0)
    j = jax.lax.rem(i, NBLKB)
    l = jax.lax.div(i, NBLKB)

    @pl.when(i == 0)
    def _():
        hcur_ref[...] = jnp.dot(x1_ref[...], W2_ref[...],
                                preferred_element_type=jnp.float32
                                ).astype(jnp.bfloat16)

    @pl.when(i == NBLKB)
    def _():
        hcur_ref[...] = h3_ref[...]

    y = jnp.dot(adjb_ref[...], hcur_ref[...],
                preferred_element_type=jnp.float32)
    y = y + jnp.where(l == 0, b2_ref[...], b3_ref[...])
    yr = jnp.maximum(y, 0.0)

    @pl.when(l == 0)
    def _():
        h3_ref[pl.ds(j * BLKB, BLKB), :] = jnp.dot(
            yr, W3_ref[...],
            preferred_element_type=jnp.float32).astype(jnp.bfloat16)

        @pl.when(j == 0)
        def _():
            acc2_ref[...] = yr

        @pl.when(j != 0)
        def _():
            acc2_ref[...] = jnp.maximum(acc2_ref[...], yr)

    @pl.when(l == 1)
    def _():
        @pl.when(j == 0)
        def _():
            acc3_ref[...] = y

        @pl.when(j != 0)
        def _():
            acc3_ref[...] = jnp.maximum(acc3_ref[...], y)

    @pl.when(i == 2 * NBLKB - 1)
    def _():
        o2 = jnp.max(acc2_ref[...], axis=0, keepdims=True)
        o3 = jnp.max(acc3_ref[...], axis=0, keepdims=True)
        logits = (jnp.sum(linW_ref[:, 0:NHID] * o1_ref[...], axis=1)
                  + jnp.sum(linW_ref[:, NHID:2 * NHID] * o2, axis=1)
                  + jnp.sum(linW_ref[:, 2 * NHID:] * o3, axis=1)
                  + linb_ref[0, :])
        z = logits - jnp.max(logits)
        out_ref[0, :] = z - jnp.log(jnp.sum(jnp.exp(z)))


def kernel(x, adj, W1, b1, W2, b2, W3, b3, linW, linb):
    full = lambda shape: pl.BlockSpec(shape, lambda i: (0, 0))

    adj_bf16, x1, o1 = pl.pallas_call(
        _layer1_kernel,
        grid=(NBLKA,),
        in_specs=[
            pl.BlockSpec((BLKA, N), lambda j: (j, 0)),
            full((N, NFEAT)),
            full((NFEAT, NHID)),
            full((1, NHID)),
        ],
        out_specs=[
            pl.BlockSpec((BLKA, N), lambda j: (j, 0)),
            pl.BlockSpec((BLKA, NHID), lambda j: (j, 0)),
            pl.BlockSpec((1, NHID), lambda j: (0, 0)),
        ],
        out_shape=[
            jax.ShapeDtypeStruct((N, N), jnp.bfloat16),
            jax.ShapeDtypeStruct((N, NHID), jnp.bfloat16),
            jax.ShapeDtypeStruct((1, NHID), jnp.float32),
        ],
        scratch_shapes=[
            pltpu.VMEM((N, NHID), jnp.bfloat16),
            pltpu.VMEM((BLKA, NHID), jnp.float32),
        ],
        compiler_params=pltpu.CompilerParams(
            dimension_semantics=("arbitrary",)),
    )(adj, x, W1, b1.reshape(1, -1))

    out = pl.pallas_call(
        _layer23_kernel,
        grid=(2 * NBLKB,),
        in_specs=[
            pl.BlockSpec((BLKB, N), lambda i: (jax.lax.rem(i, NBLKB), 0)),
            full((N, NHID)),
            full((NHID, NHID)),
            full((NHID, NHID)),
            full((1, NHID)),
            full((1, NHID)),
            full((NCLASS, 3 * NHID)),
            full((1, NCLASS)),
            full((1, NHID)),
        ],
        out_specs=pl.BlockSpec((1, NCLASS), lambda i: (0, 0)),
        out_shape=jax.ShapeDtypeStruct((1, NCLASS), jnp.float32),
        scratch_shapes=[
            pltpu.VMEM((N, NHID), jnp.bfloat16),
            pltpu.VMEM((N, NHID), jnp.bfloat16),
            pltpu.VMEM((BLKB, NHID), jnp.float32),
            pltpu.VMEM((BLKB, NHID), jnp.float32),
        ],
        compiler_params=pltpu.CompilerParams(
            dimension_semantics=("arbitrary",)),
    )(adj_bf16, x1, W2, W3, b2.reshape(1, -1), b3.reshape(1, -1), linW,
      linb.reshape(1, -1), o1)
    return out.reshape(NCLASS)
